# Initial kernel scaffold; baseline (speedup 1.0000x reference)
import jax
import jax.numpy as jnp
from jax.experimental import pallas as pl


def kernel(features):
    raise NotImplementedError("write your pallas kernel here")


# speedup vs baseline: 6.2618x; 6.2618x over previous
import functools

import jax
import jax.numpy as jnp
from jax.experimental import pallas as pl

_MIN_K = 16
_EPS = 1e-10
_BIG = 3.4e38


def _soft_knn_block(fr_ref, fa_ref, out_ref, *, n_bisect):
    fr = fr_ref[...]
    fa = fa_ref[...]
    r = fr.shape[0]
    n = fa.shape[0]

    sqr = jnp.sum(fr * fr, axis=1)
    sqa = jnp.sum(fa * fa, axis=1)
    dot = jax.lax.dot_general(
        fr, fa, (((1,), (1,)), ((), ())),
        preferred_element_type=jnp.float32)
    d2 = jnp.maximum(sqr[:, None] + sqa[None, :] - 2.0 * dot, 0.0)

    c = jnp.min(d2.reshape(r, n // 128, 128), axis=2)

    def _mask_min(_, cc):
        m = jnp.min(cc, axis=1, keepdims=True)
        return jnp.where(cc == m, _BIG, cc)

    c = jax.lax.fori_loop(0, _MIN_K, _mask_min, c)
    ub = jnp.min(c, axis=1)

    kk = jnp.float32(_MIN_K + 1)

    def _bisect(_, carry):
        lo, hi = carry
        mid = 0.5 * (lo + hi)
        cnt = jnp.sum((d2 <= mid[:, None]).astype(jnp.float32), axis=1)
        ge = cnt >= kk
        return jnp.where(ge, lo, mid), jnp.where(ge, mid, hi)

    lo0 = jnp.zeros_like(ub)
    _, hi = jax.lax.fori_loop(0, n_bisect, _bisect, (lo0, ub))

    thr = jnp.sqrt(hi)
    dist = jnp.sqrt(d2)
    w = jnp.maximum(thr[:, None] - dist + _EPS, 0.0)
    norm = jnp.sum(w, axis=1, keepdims=True)
    out_ref[...] = w / jnp.maximum(norm, 1e-12)


def kernel(features):
    n, d = features.shape
    block_r = 256
    grid = (n // block_r,)
    return pl.pallas_call(
        functools.partial(_soft_knn_block, n_bisect=16),
        grid=grid,
        in_specs=[
            pl.BlockSpec((block_r, d), lambda i: (i, 0)),
            pl.BlockSpec((n, d), lambda i: (0, 0)),
        ],
        out_specs=pl.BlockSpec((block_r, n), lambda i: (i, 0)),
        out_shape=jax.ShapeDtypeStruct((n, n), jnp.float32),
    )(features, features)


# speedup vs baseline: 23.3921x; 3.7357x over previous
import functools

import jax
import jax.numpy as jnp
from jax.experimental import pallas as pl

_MIN_K = 16
_EPS = 1e-10
_BIG = 3.4e38


def _soft_knn_block(fr_ref, fa_ref, out_ref, *, n_bisect):
    fr = fr_ref[...]
    fa = fa_ref[...]
    r = fr.shape[0]
    n = fa.shape[0]

    sqr = jnp.sum(fr * fr, axis=1, keepdims=True)
    sqa = jnp.sum(fa * fa, axis=1, keepdims=True)
    dot = jax.lax.dot_general(
        fr, fa, (((1,), (1,)), ((), ())),
        preferred_element_type=jnp.float32)
    d2 = jnp.maximum(sqr + sqa.reshape(1, n) - 2.0 * dot, 0.0)

    c = jnp.min(d2.reshape(r, n // 128, 128), axis=1)

    def _mask_min(_, cc):
        m = jnp.min(cc, axis=1, keepdims=True)
        return jnp.where(cc == m, _BIG, cc)

    c = jax.lax.fori_loop(0, _MIN_K, _mask_min, c)
    ub = jnp.min(c, axis=1, keepdims=True)

    kk = jnp.float32(_MIN_K + 1)

    def _bisect(_, carry):
        lo, hi = carry
        mid = 0.5 * (lo + hi)
        cnt = jnp.sum(jnp.where(d2 <= mid, 1.0, 0.0), axis=1, keepdims=True)
        ge = cnt >= kk
        return jnp.where(ge, lo, mid), jnp.where(ge, mid, hi)

    lo0 = jnp.zeros_like(ub)
    _, hi = jax.lax.fori_loop(0, n_bisect, _bisect, (lo0, ub))

    thr = jnp.sqrt(hi)
    dist = jnp.sqrt(d2)
    w = jnp.maximum(thr - dist + _EPS, 0.0)
    norm = jnp.sum(w, axis=1, keepdims=True)
    out_ref[...] = w / jnp.maximum(norm, 1e-12)


def kernel(features):
    n, d = features.shape
    block_r = 256
    grid = (n // block_r,)
    return pl.pallas_call(
        functools.partial(_soft_knn_block, n_bisect=12),
        grid=grid,
        in_specs=[
            pl.BlockSpec((block_r, d), lambda i: (i, 0)),
            pl.BlockSpec((n, d), lambda i: (0, 0)),
        ],
        out_specs=pl.BlockSpec((block_r, n), lambda i: (i, 0)),
        out_shape=jax.ShapeDtypeStruct((n, n), jnp.float32),
    )(features, features)


# speedup vs baseline: 25.3493x; 1.0837x over previous
import functools

import jax
import jax.numpy as jnp
from jax.experimental import pallas as pl

_MIN_K = 16
_EPS = 1e-10
_BIG = 3.4e38


def _soft_knn_block(fr_ref, fa_ref, out_ref, *, n_bisect):
    fr = fr_ref[...]
    fa = fa_ref[...]
    r = fr.shape[0]
    n = fa.shape[0]

    sqr = jnp.sum(fr * fr, axis=1, keepdims=True)
    sqa = jnp.sum(fa * fa, axis=1, keepdims=True)
    dot = jax.lax.dot_general(
        fr, fa, (((1,), (1,)), ((), ())),
        preferred_element_type=jnp.float32)
    d2 = jnp.maximum(sqr + sqa.reshape(1, n) - 2.0 * dot, 0.0)

    c = jnp.min(d2.reshape(r, n // 128, 128), axis=1)

    def _mask_min(_, cc):
        m = jnp.min(cc, axis=1, keepdims=True)
        return jnp.where(cc == m, _BIG, cc)

    c = jax.lax.fori_loop(0, _MIN_K, _mask_min, c)
    ub = jnp.min(c, axis=1, keepdims=True)

    kk = jnp.float32(_MIN_K + 1)

    def _bisect(_, carry):
        lo, hi = carry
        mid = 0.5 * (lo + hi)
        cnt = jnp.sum(jnp.where(d2 <= mid, 1.0, 0.0), axis=1, keepdims=True)
        ge = cnt >= kk
        return jnp.where(ge, lo, mid), jnp.where(ge, mid, hi)

    lo0 = jnp.zeros_like(ub)
    _, hi = jax.lax.fori_loop(0, n_bisect, _bisect, (lo0, ub))

    thr = jnp.sqrt(hi)
    dist = jnp.sqrt(d2)
    w = jnp.maximum(thr - dist + _EPS, 0.0)
    norm = jnp.sum(w, axis=1, keepdims=True)
    out_ref[...] = w / jnp.maximum(norm, 1e-12)


def kernel(features):
    n, d = features.shape
    block_r = 256
    grid = (n // block_r,)
    return pl.pallas_call(
        functools.partial(_soft_knn_block, n_bisect=10),
        grid=grid,
        in_specs=[
            pl.BlockSpec((block_r, d), lambda i: (i, 0)),
            pl.BlockSpec((n, d), lambda i: (0, 0)),
        ],
        out_specs=pl.BlockSpec((block_r, n), lambda i: (i, 0)),
        out_shape=jax.ShapeDtypeStruct((n, n), jnp.float32),
    )(features, features)


# speedup vs baseline: 26.6966x; 1.0531x over previous
import functools

import jax
import jax.numpy as jnp
from jax.experimental import pallas as pl

_MIN_K = 16
_EPS = 1e-10
_BIG = 3.4e38


def _soft_knn_block(fr_ref, fa_ref, out_ref, *, n_bisect):
    fr = fr_ref[...]
    fa = fa_ref[...]
    r = fr.shape[0]
    n = fa.shape[0]

    sqr = jnp.sum(fr * fr, axis=1, keepdims=True)
    sqa = jnp.sum(fa * fa, axis=1, keepdims=True)
    dot = jax.lax.dot_general(
        fr, fa, (((1,), (1,)), ((), ())),
        preferred_element_type=jnp.float32)
    d2 = jnp.maximum(sqr + sqa.reshape(1, n) - 2.0 * dot, 0.0)

    c = jnp.min(d2.reshape(r, n // 128, 128), axis=1)

    def _mask_min(_, cc):
        m = jnp.min(cc, axis=1, keepdims=True)
        return jnp.where(cc == m, _BIG, cc)

    c = jax.lax.fori_loop(0, _MIN_K, _mask_min, c)
    ub = jnp.min(c, axis=1, keepdims=True)

    kk = jnp.float32(_MIN_K + 1)

    def _bisect(_, carry):
        lo, hi = carry
        mid = 0.5 * (lo + hi)
        cnt = jnp.sum(jnp.where(d2 <= mid, 1.0, 0.0), axis=1, keepdims=True)
        ge = cnt >= kk
        return jnp.where(ge, lo, mid), jnp.where(ge, mid, hi)

    lo0 = jnp.zeros_like(ub)
    lo, hi = jax.lax.fori_loop(0, n_bisect, _bisect, (lo0, ub))

    thr = jnp.sqrt(0.5 * (lo + hi))
    dist = jnp.sqrt(d2)
    w = jnp.maximum(thr - dist + _EPS, 0.0)
    norm = jnp.sum(w, axis=1, keepdims=True)
    out_ref[...] = w / jnp.maximum(norm, 1e-12)


def kernel(features):
    n, d = features.shape
    block_r = 512
    grid = (n // block_r,)
    return pl.pallas_call(
        functools.partial(_soft_knn_block, n_bisect=10),
        grid=grid,
        in_specs=[
            pl.BlockSpec((block_r, d), lambda i: (i, 0)),
            pl.BlockSpec((n, d), lambda i: (0, 0)),
        ],
        out_specs=pl.BlockSpec((block_r, n), lambda i: (i, 0)),
        out_shape=jax.ShapeDtypeStruct((n, n), jnp.float32),
    )(features, features)


# speedup vs baseline: 37.9642x; 1.4221x over previous
import functools

import jax
import jax.numpy as jnp
from jax.experimental import pallas as pl

_MIN_K = 16
_EPS = 1e-10


def _soft_knn_block(fr_ref, fa_ref, out_ref, *, n_bisect):
    fr = fr_ref[...]
    fa = fa_ref[...]
    r = fr.shape[0]
    n = fa.shape[0]

    sqr = jnp.sum(fr * fr, axis=1, keepdims=True)
    sqa = jnp.sum(fa * fa, axis=1, keepdims=True)
    dot = jax.lax.dot_general(
        fr, fa, (((1,), (1,)), ((), ())),
        preferred_element_type=jnp.float32)
    d2 = jnp.maximum(sqr + sqa.reshape(1, n) - 2.0 * dot, 0.0)

    slices = [d2[:, j * 128:(j + 1) * 128] for j in range(n // 128)]
    while len(slices) > 1:
        half = len(slices) // 2
        slices = [jnp.minimum(slices[i], slices[i + half])
                  for i in range(half)] + slices[2 * half:]
    c = slices[0]

    kk = jnp.float32(_MIN_K + 1)

    def _bisect_c(_, carry):
        lo, hi = carry
        mid = 0.5 * (lo + hi)
        cnt = jnp.sum(jnp.where(c <= mid, 1.0, 0.0), axis=1, keepdims=True)
        ge = cnt >= kk
        return jnp.where(ge, lo, mid), jnp.where(ge, mid, hi)

    cmax = jnp.max(c, axis=1, keepdims=True)
    _, ub = jax.lax.fori_loop(0, 12, _bisect_c, (jnp.zeros_like(cmax), cmax))


    def _bisect(_, carry):
        lo, hi = carry
        mid = 0.5 * (lo + hi)
        cnt = jnp.sum(jnp.where(d2 <= mid, 1.0, 0.0), axis=1, keepdims=True)
        ge = cnt >= kk
        return jnp.where(ge, lo, mid), jnp.where(ge, mid, hi)

    lo0 = jnp.zeros_like(ub)
    lo, hi = jax.lax.fori_loop(0, n_bisect, _bisect, (lo0, ub))

    thr = jnp.sqrt(0.5 * (lo + hi))
    dist = jnp.sqrt(d2)
    w = jnp.maximum(thr - dist + _EPS, 0.0)
    norm = jnp.sum(w, axis=1, keepdims=True)
    out_ref[...] = w / jnp.maximum(norm, 1e-12)


def kernel(features):
    n, d = features.shape
    block_r = 512
    grid = (n // block_r,)
    return pl.pallas_call(
        functools.partial(_soft_knn_block, n_bisect=9),
        grid=grid,
        in_specs=[
            pl.BlockSpec((block_r, d), lambda i: (i, 0)),
            pl.BlockSpec((n, d), lambda i: (0, 0)),
        ],
        out_specs=pl.BlockSpec((block_r, n), lambda i: (i, 0)),
        out_shape=jax.ShapeDtypeStruct((n, n), jnp.float32),
    )(features, features)


# speedup vs baseline: 38.2411x; 1.0073x over previous
import functools

import jax
import jax.numpy as jnp
from jax.experimental import pallas as pl

_MIN_K = 16
_EPS = 1e-10


def _soft_knn_block(fr_ref, fa_ref, out_ref, *, n_bisect):
    fr = fr_ref[...]
    fa = fa_ref[...]
    r = fr.shape[0]
    n = fa.shape[0]

    sqr = jnp.sum(fr * fr, axis=1, keepdims=True)
    sqa = jnp.sum(fa * fa, axis=1, keepdims=True)
    dot = jax.lax.dot_general(
        fr * -2.0, fa, (((1,), (1,)), ((), ())),
        preferred_element_type=jnp.float32)
    d2 = jnp.maximum(sqr + sqa.reshape(1, n) + dot, 0.0)

    slices = [d2[:, j * 128:(j + 1) * 128] for j in range(n // 128)]
    while len(slices) > 1:
        half = len(slices) // 2
        slices = [jnp.minimum(slices[i], slices[i + half])
                  for i in range(half)] + slices[2 * half:]
    c = slices[0]

    kk = jnp.float32(_MIN_K + 1)

    def _bisect_c(_, carry):
        lo, hi = carry
        mid = 0.5 * (lo + hi)
        cnt = jnp.sum(jnp.where(c <= mid, 1.0, 0.0), axis=1, keepdims=True)
        ge = cnt >= kk
        return jnp.where(ge, lo, mid), jnp.where(ge, mid, hi)

    cmax = jnp.max(c, axis=1, keepdims=True)
    _, ub = jax.lax.fori_loop(0, 12, _bisect_c, (jnp.zeros_like(cmax), cmax))


    def _bisect(_, carry):
        lo, hi = carry
        mid = 0.5 * (lo + hi)
        cnt = jnp.sum(jnp.where(d2 <= mid, 1.0, 0.0), axis=1, keepdims=True)
        ge = cnt >= kk
        return jnp.where(ge, lo, mid), jnp.where(ge, mid, hi)

    lo0 = jnp.zeros_like(ub)
    lo, hi = jax.lax.fori_loop(0, n_bisect, _bisect, (lo0, ub))

    thr = jnp.sqrt(0.5 * (lo + hi))
    dist = jnp.sqrt(d2)
    w = jnp.maximum(thr - dist + _EPS, 0.0)
    norm = jnp.sum(w, axis=1, keepdims=True)
    out_ref[...] = w / jnp.maximum(norm, 1e-12)


def kernel(features):
    n, d = features.shape
    block_r = 512
    grid = (n // block_r,)
    return pl.pallas_call(
        functools.partial(_soft_knn_block, n_bisect=9),
        grid=grid,
        in_specs=[
            pl.BlockSpec((block_r, d), lambda i: (i, 0)),
            pl.BlockSpec((n, d), lambda i: (0, 0)),
        ],
        out_specs=pl.BlockSpec((block_r, n), lambda i: (i, 0)),
        out_shape=jax.ShapeDtypeStruct((n, n), jnp.float32),
    )(features, features)


# speedup vs baseline: 40.4283x; 1.0572x over previous
import functools

import jax
import jax.numpy as jnp
from jax.experimental import pallas as pl

_MIN_K = 16
_EPS = 1e-10


def _soft_knn_block(fr_ref, fa_ref, out_ref, *, n_bisect):
    fr = fr_ref[...]
    fa = fa_ref[...]
    r = fr.shape[0]
    n = fa.shape[0]

    sqr = jnp.sum(fr * fr, axis=1, keepdims=True)
    sqa = jnp.sum(fa * fa, axis=1, keepdims=True)
    dot = jax.lax.dot_general(
        fr * -2.0, fa, (((1,), (1,)), ((), ())),
        preferred_element_type=jnp.float32)
    d2 = jnp.maximum(sqr + sqa.reshape(1, n) + dot, 0.0)

    slices = [d2[:, j * 128:(j + 1) * 128] for j in range(n // 128)]
    while len(slices) > 1:
        half = len(slices) // 2
        slices = [jnp.minimum(slices[i], slices[i + half])
                  for i in range(half)] + slices[2 * half:]
    c = slices[0]

    kk = jnp.float32(_MIN_K + 1)

    def _bisect_c(_, carry):
        lo, hi = carry
        mid = 0.5 * (lo + hi)
        cnt = jnp.sum(jnp.where(c <= mid, 1.0, 0.0), axis=1, keepdims=True)
        ge = cnt >= kk
        return jnp.where(ge, lo, mid), jnp.where(ge, mid, hi)

    cmax = jnp.max(c, axis=1, keepdims=True)
    _, ub = jax.lax.fori_loop(0, 12, _bisect_c, (jnp.zeros_like(cmax), cmax))


    def _bisect(_, carry):
        lo, hi = carry
        mid = 0.5 * (lo + hi)
        cnt = jnp.sum(jnp.where(d2 <= mid, 1.0, 0.0), axis=1, keepdims=True)
        ge = cnt >= kk
        return jnp.where(ge, lo, mid), jnp.where(ge, mid, hi)

    lo0 = jnp.zeros_like(ub)
    lo, hi = jax.lax.fori_loop(0, n_bisect, _bisect, (lo0, ub))

    thr = jnp.sqrt(0.5 * (lo + hi))
    dist = d2 * jax.lax.rsqrt(jnp.maximum(d2, 1e-37))
    w = jnp.maximum(thr - dist + _EPS, 0.0)
    norm = jnp.sum(w, axis=1, keepdims=True)
    out_ref[...] = w / jnp.maximum(norm, 1e-12)


def kernel(features):
    n, d = features.shape
    block_r = 512
    grid = (n // block_r,)
    return pl.pallas_call(
        functools.partial(_soft_knn_block, n_bisect=9),
        grid=grid,
        in_specs=[
            pl.BlockSpec((block_r, d), lambda i: (i, 0)),
            pl.BlockSpec((n, d), lambda i: (0, 0)),
        ],
        out_specs=pl.BlockSpec((block_r, n), lambda i: (i, 0)),
        out_shape=jax.ShapeDtypeStruct((n, n), jnp.float32),
    )(features, features)


# speedup vs baseline: 40.7167x; 1.0071x over previous
import functools

import jax
import jax.numpy as jnp
from jax.experimental import pallas as pl
from jax.experimental.pallas import tpu as pltpu

_MIN_K = 16
_EPS = 1e-10


def _soft_knn_block(fr_ref, fa_ref, out_ref, sqa_ref, *, n_bisect):
    fr = fr_ref[...]
    fa = fa_ref[...]
    r = fr.shape[0]
    n = fa.shape[0]

    sqa = jnp.sum(fa * fa, axis=1, keepdims=True)
    sqa_ref[...] = sqa
    sqr = sqa_ref[pl.ds(pl.program_id(0) * r, r), :]
    dot = jax.lax.dot_general(
        fr * -2.0, fa, (((1,), (1,)), ((), ())),
        preferred_element_type=jnp.float32)
    d2 = jnp.maximum(sqr + sqa.reshape(1, n) + dot, 0.0)

    slices = [d2[:, j * 128:(j + 1) * 128] for j in range(n // 128)]
    while len(slices) > 1:
        half = len(slices) // 2
        slices = [jnp.minimum(slices[i], slices[i + half])
                  for i in range(half)] + slices[2 * half:]
    c = slices[0]

    kk = jnp.float32(_MIN_K + 1)

    def _bisect_c(_, carry):
        lo, hi = carry
        mid = 0.5 * (lo + hi)
        cnt = jnp.sum(jnp.where(c <= mid, 1.0, 0.0), axis=1, keepdims=True)
        ge = cnt >= kk
        return jnp.where(ge, lo, mid), jnp.where(ge, mid, hi)

    cmax = jnp.max(c, axis=1, keepdims=True)
    _, ub = jax.lax.fori_loop(0, 10, _bisect_c, (jnp.zeros_like(cmax), cmax))


    def _bisect(_, carry):
        lo, hi = carry
        mid = 0.5 * (lo + hi)
        cnt = jnp.sum(jnp.where(d2 <= mid, 1.0, 0.0), axis=1, keepdims=True)
        ge = cnt >= kk
        return jnp.where(ge, lo, mid), jnp.where(ge, mid, hi)

    lo0 = jnp.zeros_like(ub)
    lo, hi = jax.lax.fori_loop(0, n_bisect, _bisect, (lo0, ub))

    thr = jnp.sqrt(0.5 * (lo + hi))
    dist = d2 * jax.lax.rsqrt(jnp.maximum(d2, 1e-37))
    w = jnp.maximum(thr - dist + _EPS, 0.0)
    norm = jnp.sum(w, axis=1, keepdims=True)
    out_ref[...] = w / jnp.maximum(norm, 1e-12)


def kernel(features):
    n, d = features.shape
    block_r = 512
    grid = (n // block_r,)
    return pl.pallas_call(
        functools.partial(_soft_knn_block, n_bisect=9),
        grid=grid,
        in_specs=[
            pl.BlockSpec((block_r, d), lambda i: (i, 0)),
            pl.BlockSpec((n, d), lambda i: (0, 0)),
        ],
        out_specs=pl.BlockSpec((block_r, n), lambda i: (i, 0)),
        out_shape=jax.ShapeDtypeStruct((n, n), jnp.float32),
        scratch_shapes=[pltpu.VMEM((n, 1), jnp.float32)],
    )(features, features)
